# baseline (speedup 1.0000x reference)
import functools

import jax
import jax.numpy as jnp
from jax import lax
from jax.experimental import pallas as pl
from jax.experimental.pallas import tpu as pltpu
from jax.experimental.pallas import tpu_sc as plsc

_NC = 2
_NS = 16
_NW = _NC * _NS
_L = 16


@functools.cache
def _build_transpose(n_rows, dim, blk):
    n_main = n_rows // blk
    n_blk = -(-n_rows // blk)
    has_tail = n_blk != n_main

    def body(t_ref, tail_ref, o_ref):
        j = pl.program_id(0)
        x = t_ref[...]
        if has_tail:
            x = jnp.where(j == n_main, tail_ref[...], x)
        o_ref[...] = x.T

    call = pl.pallas_call(
        body,
        grid=(n_blk,),
        in_specs=[
            pl.BlockSpec((dim, blk),
                         lambda j: (0, jnp.minimum(j, n_main - 1))),
            pl.BlockSpec((dim, blk), lambda j: (0, 0)),
        ],
        out_specs=pl.BlockSpec((blk, dim), lambda j: (j, 0)),
        out_shape=jax.ShapeDtypeStruct((n_blk * blk, dim), jnp.float32),
        compiler_params=pltpu.CompilerParams(
            dimension_semantics=("arbitrary",)),
    )

    def apply(tab_t):
        tail = tab_t[:, n_main * blk:]
        tail = jnp.pad(tail, ((0, 0), (0, n_blk * blk - n_rows)))
        return call(tab_t, tail)

    return apply


@functools.cache
def _build(B, n_cond, dim, n_cat):
    total = B * n_cond
    per_w = total // _NW
    chunk = 1664 if per_w % 1664 == 0 else per_w
    n_chunks = per_w // chunk
    reps = chunk // n_cond
    mesh = plsc.VectorSubcoreMesh(core_axis_name="c", subcore_axis_name="s")

    @functools.partial(
        pl.kernel,
        out_type=jax.ShapeDtypeStruct((total, dim), jnp.float32),
        mesh=mesh,
        scratch_types=[
            pltpu.VMEM((chunk,), jnp.int32),
            pltpu.VMEM((chunk,), jnp.int32),
            pltpu.VMEM((chunk,), jnp.int32),
            pltpu.VMEM((chunk,), jnp.int32),
            pltpu.VMEM((chunk, dim), jnp.float32),
            pltpu.VMEM((chunk, dim), jnp.float32),
            pltpu.VMEM((chunk,), jnp.int32),
            pltpu.VMEM((n_cond, dim), jnp.float32),
            pltpu.SemaphoreType.DMA,
            pltpu.SemaphoreType.DMA,
            pltpu.SemaphoreType.DMA,
            pltpu.SemaphoreType.DMA,
            pltpu.SemaphoreType.DMA,
            pltpu.SemaphoreType.DMA,
            pltpu.SemaphoreType.DMA,
            pltpu.SemaphoreType.DMA,
        ],
        compiler_params=pltpu.CompilerParams(use_tc_tiling_on_sc=False),
    )
    def run(ids_hbm, cond_hbm, cat_hbm, offs_hbm, out_hbm,
            ix0, ix1, ix2, ix3, rw0, rw1, offs_v, bias_v,
            si0, si1, si2, si3, sg0, sg1, so0, so1):
        wid = lax.axis_index("s") * _NC + lax.axis_index("c")
        base = wid * per_w
        idxs = [ix0, ix1, ix2, ix3]
        rows = [rw0, rw1]
        sid = [si0, si1, si2, si3]
        sg = [sg0, sg1]
        so = [so0, so1]
        pltpu.sync_copy(offs_hbm, offs_v)
        pltpu.sync_copy(cond_hbm.at[pl.ds(1, n_cond)], bias_v)

        descs = {}

        def s_ids(i):
            descs["ids", i] = pltpu.async_copy(
                ids_hbm.at[pl.ds(base + i * chunk, chunk)], idxs[i % 4],
                sid[i % 4])

        def s_gat(i):
            descs["gat", i] = pltpu.async_copy(
                cat_hbm.at[idxs[i % 4]], rows[i % 2], sg[i % 2])

        def s_out(i):
            descs["out", i] = pltpu.async_copy(
                rows[i % 2], out_hbm.at[pl.ds(base + i * chunk, chunk)],
                so[i % 2])

        def add_offs(i):
            ix = idxs[i % 4]

            def body(k, c):
                sl = pl.ds(k * _L, _L)
                ix[sl] = ix[sl] + offs_v[sl]
                return c

            lax.fori_loop(0, chunk // _L, body, 0)

        def add_bias(i):
            r = rows[i % 2]

            def body(rep, c):
                r0 = rep * n_cond
                for rb in range(n_cond):
                    for h in range(dim // _L):
                        sl = pl.ds(h * _L, _L)
                        r[r0 + rb, sl] = r[r0 + rb, sl] + bias_v[rb, sl]
                return c

            lax.fori_loop(0, reps, body, 0)

        s_ids(0)
        s_ids(1)
        descs["ids", 0].wait()
        add_offs(0)
        s_gat(0)
        if n_chunks > 2:
            s_ids(2)
        for i in range(n_chunks):
            if i + 1 < n_chunks:
                descs["ids", i + 1].wait()
                add_offs(i + 1)
                if i >= 1:
                    descs["out", i - 1].wait()
                s_gat(i + 1)
                if i + 3 < n_chunks:
                    s_ids(i + 3)
            descs["gat", i].wait()
            add_bias(i)
            s_out(i)
        if n_chunks >= 2:
            descs["out", n_chunks - 2].wait()
        descs["out", n_chunks - 1].wait()

    return run, chunk


def kernel(cat_ids, cond_table, cat_table):
    B, n_cond = cat_ids.shape
    n_rows, dim = cat_table.shape
    n_cat = n_rows // n_cond
    run, chunk = _build(B, n_cond, dim, n_cat)
    tab_rm = _build_transpose(n_rows, dim, 512)(cat_table.T)
    ids_flat = cat_ids.reshape(-1).astype(jnp.int32)
    offs = jnp.tile(jnp.arange(n_cond, dtype=jnp.int32) * n_cat,
                    chunk // n_cond)
    out = run(ids_flat, cond_table, tab_rm, offs)
    return out.reshape(B, n_cond, dim)


# speedup vs baseline: 2.0801x; 2.0801x over previous
import functools

import jax
import jax.numpy as jnp
from jax import lax
from jax.experimental import pallas as pl
from jax.experimental.pallas import tpu as pltpu
from jax.experimental.pallas import tpu_sc as plsc

_NC = 2
_NS = 16
_NW = _NC * _NS
_L = 16


@functools.cache
def _build_transpose(n_rows, dim, blk):
    n_main = n_rows // blk
    n_blk = -(-n_rows // blk)
    has_tail = n_blk != n_main

    def body(t_ref, tail_ref, o_ref):
        j = pl.program_id(0)
        x = t_ref[...]
        if has_tail:
            x = jnp.where(j == n_main, tail_ref[...], x)
        o_ref[...] = x.T

    call = pl.pallas_call(
        body,
        grid=(n_blk,),
        in_specs=[
            pl.BlockSpec((dim, blk),
                         lambda j: (0, jnp.minimum(j, n_main - 1))),
            pl.BlockSpec((dim, blk), lambda j: (0, 0)),
        ],
        out_specs=pl.BlockSpec((blk, dim), lambda j: (j, 0)),
        out_shape=jax.ShapeDtypeStruct((n_blk * blk, dim), jnp.float32),
        compiler_params=pltpu.CompilerParams(
            dimension_semantics=("arbitrary",)),
    )

    def apply(tab_t):
        tail = tab_t[:, n_main * blk:]
        tail = jnp.pad(tail, ((0, 0), (0, n_blk * blk - n_rows)))
        return call(tab_t, tail)

    return apply


@functools.cache
def _build(B, n_cond, dim, n_cat):
    total = B * n_cond
    per_w = total // _NW
    chunk = 1664 if per_w % 1664 == 0 else per_w
    n_chunks = per_w // chunk
    reps = chunk // n_cond
    mesh = plsc.VectorSubcoreMesh(core_axis_name="c", subcore_axis_name="s")

    @functools.partial(
        pl.kernel,
        out_type=jax.ShapeDtypeStruct((total, dim), jnp.float32),
        mesh=mesh,
        scratch_types=[
            pltpu.VMEM((chunk,), jnp.int32),
            pltpu.VMEM((chunk,), jnp.int32),
            pltpu.VMEM((chunk,), jnp.int32),
            pltpu.VMEM((chunk,), jnp.int32),
            pltpu.VMEM((chunk, dim), jnp.float32),
            pltpu.VMEM((chunk, dim), jnp.float32),
            pltpu.VMEM((chunk,), jnp.int32),
            pltpu.VMEM((n_cond, dim), jnp.float32),
            pltpu.SemaphoreType.DMA,
            pltpu.SemaphoreType.DMA,
            pltpu.SemaphoreType.DMA,
            pltpu.SemaphoreType.DMA,
            pltpu.SemaphoreType.DMA,
            pltpu.SemaphoreType.DMA,
            pltpu.SemaphoreType.DMA,
            pltpu.SemaphoreType.DMA,
        ],
        compiler_params=pltpu.CompilerParams(use_tc_tiling_on_sc=False),
    )
    def run(ids_hbm, cond_hbm, cat_hbm, offs_hbm, out_hbm,
            ix0, ix1, ix2, ix3, rw0, rw1, offs_v, bias_v,
            si0, si1, si2, si3, sg0, sg1, so0, so1):
        wid = lax.axis_index("s") * _NC + lax.axis_index("c")
        base = wid * per_w
        idxs = [ix0, ix1, ix2, ix3]
        rows = [rw0, rw1]
        sid = [si0, si1, si2, si3]
        sg = [sg0, sg1]
        so = [so0, so1]
        pltpu.sync_copy(offs_hbm, offs_v)
        pltpu.sync_copy(cond_hbm.at[pl.ds(1, n_cond)], bias_v)

        descs = {}

        def s_ids(i):
            descs["ids", i] = pltpu.async_copy(
                ids_hbm.at[pl.ds(base + i * chunk, chunk)], idxs[i % 4],
                sid[i % 4])

        def s_gat(i):
            descs["gat", i] = pltpu.async_copy(
                cat_hbm.at[idxs[i % 4]], rows[i % 2], sg[i % 2])

        def s_out(i):
            descs["out", i] = pltpu.async_copy(
                rows[i % 2], out_hbm.at[pl.ds(base + i * chunk, chunk)],
                so[i % 2])

        def add_offs(i):
            ix = idxs[i % 4]

            def body(k, c):
                sl = pl.ds(k * _L, _L)
                ix[sl] = ix[sl] + offs_v[sl]
                return c

            lax.fori_loop(0, chunk // _L, body, 0)

        def add_bias(i):
            r = rows[i % 2]

            def body(rep, c):
                r0 = rep * n_cond
                for rb in range(n_cond):
                    for h in range(dim // _L):
                        sl = pl.ds(h * _L, _L)
                        r[r0 + rb, sl] = r[r0 + rb, sl] + bias_v[rb, sl]
                return c

            lax.fori_loop(0, reps, body, 0)

        s_ids(0)
        s_ids(1)
        descs["ids", 0].wait()
        add_offs(0)
        s_gat(0)
        if n_chunks > 2:
            s_ids(2)
        for i in range(n_chunks):
            if i + 1 < n_chunks:
                descs["ids", i + 1].wait()
                add_offs(i + 1)
                if i >= 1:
                    descs["out", i - 1].wait()
                s_gat(i + 1)
                if i + 3 < n_chunks:
                    s_ids(i + 3)
            descs["gat", i].wait()
            add_bias(i)
            s_out(i)
        if n_chunks >= 2:
            descs["out", n_chunks - 2].wait()
        descs["out", n_chunks - 1].wait()

    return run, chunk


def kernel(cat_ids, cond_table, cat_table):
    B, n_cond = cat_ids.shape
    n_rows, dim = cat_table.shape
    n_cat = n_rows // n_cond
    run, chunk = _build(B, n_cond, dim, n_cat)
    tab_rm = _build_transpose(n_rows, dim, 4096)(cat_table.T)
    ids_flat = cat_ids.reshape(-1).astype(jnp.int32)
    offs = jnp.tile(jnp.arange(n_cond, dtype=jnp.int32) * n_cat,
                    chunk // n_cond)
    out = run(ids_flat, cond_table, tab_rm, offs)
    return out.reshape(B, n_cond, dim)


# speedup vs baseline: 2.3840x; 1.1461x over previous
import functools

import jax
import jax.numpy as jnp
from jax import lax
from jax.experimental import pallas as pl
from jax.experimental.pallas import tpu as pltpu
from jax.experimental.pallas import tpu_sc as plsc

_NC = 2
_NS = 16
_NW = _NC * _NS
_L = 16


@functools.cache
def _build_transpose(n_rows, dim, blk):
    n_main = n_rows // blk
    n_blk = -(-n_rows // blk)
    has_tail = n_blk != n_main

    def body(t_ref, tail_ref, o_ref):
        j = pl.program_id(0)
        x = t_ref[...]
        if has_tail:
            x = jnp.where(j == n_main, tail_ref[...], x)
        o_ref[...] = x.T

    call = pl.pallas_call(
        body,
        grid=(n_blk,),
        in_specs=[
            pl.BlockSpec((dim, blk),
                         lambda j: (0, jnp.minimum(j, n_main - 1))),
            pl.BlockSpec((dim, blk), lambda j: (0, 0)),
        ],
        out_specs=pl.BlockSpec((blk, dim), lambda j: (j, 0)),
        out_shape=jax.ShapeDtypeStruct((n_blk * blk, dim), jnp.float32),
        compiler_params=pltpu.CompilerParams(
            dimension_semantics=("arbitrary",)),
    )

    def apply(tab_t):
        tail = tab_t[:, n_main * blk:]
        tail = jnp.pad(tail, ((0, 0), (0, n_blk * blk - n_rows)))
        return call(tab_t, tail)

    return apply


@functools.cache
def _build(B, n_cond, dim, n_cat):
    total = B * n_cond
    per_w = total // _NW
    chunk = 1664 if per_w % 1664 == 0 else per_w
    n_chunks = per_w // chunk
    reps = chunk // n_cond
    mesh = plsc.VectorSubcoreMesh(core_axis_name="c", subcore_axis_name="s")

    @functools.partial(
        pl.kernel,
        out_type=jax.ShapeDtypeStruct((total, dim), jnp.float32),
        mesh=mesh,
        scratch_types=[
            pltpu.VMEM((chunk,), jnp.int32),
            pltpu.VMEM((chunk,), jnp.int32),
            pltpu.VMEM((chunk,), jnp.int32),
            pltpu.VMEM((chunk,), jnp.int32),
            pltpu.VMEM((chunk, dim), jnp.float32),
            pltpu.VMEM((chunk, dim), jnp.float32),
            pltpu.VMEM((chunk,), jnp.int32),
            pltpu.VMEM((n_cond, dim), jnp.float32),
            pltpu.SemaphoreType.DMA,
            pltpu.SemaphoreType.DMA,
            pltpu.SemaphoreType.DMA,
            pltpu.SemaphoreType.DMA,
            pltpu.SemaphoreType.DMA,
            pltpu.SemaphoreType.DMA,
            pltpu.SemaphoreType.DMA,
            pltpu.SemaphoreType.DMA,
        ],
        compiler_params=pltpu.CompilerParams(use_tc_tiling_on_sc=False),
    )
    def run(ids_hbm, cond_hbm, cat_hbm, offs_hbm, out_hbm,
            ix0, ix1, ix2, ix3, rw0, rw1, offs_v, bias_v,
            si0, si1, si2, si3, sg0, sg1, so0, so1):
        wid = lax.axis_index("s") * _NC + lax.axis_index("c")
        base = wid * per_w
        idxs = [ix0, ix1, ix2, ix3]
        rows = [rw0, rw1]
        sid = [si0, si1, si2, si3]
        sg = [sg0, sg1]
        so = [so0, so1]
        pltpu.sync_copy(offs_hbm, offs_v)
        pltpu.sync_copy(cond_hbm.at[pl.ds(1, n_cond)], bias_v)

        descs = {}

        def s_ids(i):
            descs["ids", i] = pltpu.async_copy(
                ids_hbm.at[pl.ds(base + i * chunk, chunk)], idxs[i % 4],
                sid[i % 4])

        def s_gat(i):
            descs["gat", i] = pltpu.async_copy(
                cat_hbm.at[idxs[i % 4]], rows[i % 2], sg[i % 2])

        def s_out(i):
            descs["out", i] = pltpu.async_copy(
                rows[i % 2], out_hbm.at[pl.ds(base + i * chunk, chunk)],
                so[i % 2])

        def add_offs(i):
            ix = idxs[i % 4]

            def body(k, c):
                sl = pl.ds(k * _L, _L)
                ix[sl] = ix[sl] + offs_v[sl]
                return c

            lax.fori_loop(0, chunk // _L, body, 0)

        def add_bias(i):
            r = rows[i % 2]

            def body(rep, c):
                r0 = rep * n_cond
                for rb in range(n_cond):
                    for h in range(dim // _L):
                        sl = pl.ds(h * _L, _L)
                        r[r0 + rb, sl] = r[r0 + rb, sl] + bias_v[rb, sl]
                return c

            lax.fori_loop(0, reps, body, 0)

        s_ids(0)
        s_ids(1)
        descs["ids", 0].wait()
        add_offs(0)
        s_gat(0)
        if n_chunks > 2:
            s_ids(2)
        for i in range(n_chunks):
            if i + 1 < n_chunks:
                descs["ids", i + 1].wait()
                add_offs(i + 1)
                if i >= 1:
                    descs["out", i - 1].wait()
                s_gat(i + 1)
                if i + 3 < n_chunks:
                    s_ids(i + 3)
            descs["gat", i].wait()
            add_bias(i)
            s_out(i)
        if n_chunks >= 2:
            descs["out", n_chunks - 2].wait()
        descs["out", n_chunks - 1].wait()

    return run, chunk


def kernel(cat_ids, cond_table, cat_table):
    B, n_cond = cat_ids.shape
    n_rows, dim = cat_table.shape
    n_cat = n_rows // n_cond
    run, chunk = _build(B, n_cond, dim, n_cat)
    tab_rm = _build_transpose(n_rows, dim, 16384)(cat_table.T)
    ids_flat = cat_ids.reshape(-1).astype(jnp.int32)
    offs = jnp.tile(jnp.arange(n_cond, dtype=jnp.int32) * n_cat,
                    chunk // n_cond)
    out = run(ids_flat, cond_table, tab_rm, offs)
    return out.reshape(B, n_cond, dim)


# speedup vs baseline: 2.5879x; 1.0855x over previous
import functools

import jax
import jax.numpy as jnp
from jax import lax
from jax.experimental import pallas as pl
from jax.experimental.pallas import tpu as pltpu
from jax.experimental.pallas import tpu_sc as plsc

_NC = 2
_NS = 16
_NW = _NC * _NS
_L = 16


@functools.cache
def _build(B, n_cond, dim, n_cat):
    total = B * n_cond
    per_w = total // _NW
    chunk = 1664 if per_w % 1664 == 0 else per_w
    n_chunks = per_w // chunk
    reps = chunk // n_cond
    mesh = plsc.VectorSubcoreMesh(core_axis_name="c", subcore_axis_name="s")

    @functools.partial(
        pl.kernel,
        out_type=jax.ShapeDtypeStruct((total, dim), jnp.float32),
        mesh=mesh,
        scratch_types=[
            pltpu.VMEM((chunk,), jnp.int32),
            pltpu.VMEM((chunk,), jnp.int32),
            pltpu.VMEM((chunk,), jnp.int32),
            pltpu.VMEM((chunk,), jnp.int32),
            pltpu.VMEM((chunk, dim), jnp.float32),
            pltpu.VMEM((chunk, dim), jnp.float32),
            pltpu.VMEM((chunk,), jnp.int32),
            pltpu.VMEM((n_cond, dim), jnp.float32),
            pltpu.SemaphoreType.DMA,
            pltpu.SemaphoreType.DMA,
            pltpu.SemaphoreType.DMA,
            pltpu.SemaphoreType.DMA,
            pltpu.SemaphoreType.DMA,
            pltpu.SemaphoreType.DMA,
            pltpu.SemaphoreType.DMA,
            pltpu.SemaphoreType.DMA,
        ],
        compiler_params=pltpu.CompilerParams(use_tc_tiling_on_sc=False),
    )
    def run(ids_hbm, cond_hbm, cat_hbm, offs_hbm, out_hbm,
            ix0, ix1, ix2, ix3, rw0, rw1, offs_v, bias_v,
            si0, si1, si2, si3, sg0, sg1, so0, so1):
        wid = lax.axis_index("s") * _NC + lax.axis_index("c")
        base = wid * per_w
        idxs = [ix0, ix1, ix2, ix3]
        rows = [rw0, rw1]
        sid = [si0, si1, si2, si3]
        sg = [sg0, sg1]
        so = [so0, so1]
        pltpu.sync_copy(offs_hbm, offs_v)
        pltpu.sync_copy(cond_hbm.at[pl.ds(1, n_cond)], bias_v)

        descs = {}

        def s_ids(i):
            descs["ids", i] = pltpu.async_copy(
                ids_hbm.at[pl.ds(base + i * chunk, chunk)], idxs[i % 4],
                sid[i % 4])

        def s_gat(i):
            descs["gat", i] = pltpu.async_copy(
                cat_hbm.at[idxs[i % 4]], rows[i % 2], sg[i % 2])

        def s_out(i):
            descs["out", i] = pltpu.async_copy(
                rows[i % 2], out_hbm.at[pl.ds(base + i * chunk, chunk)],
                so[i % 2])

        def add_offs(i):
            ix = idxs[i % 4]

            def body(k, c):
                sl = pl.ds(k * _L, _L)
                ix[sl] = ix[sl] + offs_v[sl]
                return c

            lax.fori_loop(0, chunk // _L, body, 0)

        def add_bias(i):
            r = rows[i % 2]

            def body(rep, c):
                r0 = rep * n_cond
                for rb in range(n_cond):
                    for h in range(dim // _L):
                        sl = pl.ds(h * _L, _L)
                        r[r0 + rb, sl] = r[r0 + rb, sl] + bias_v[rb, sl]
                return c

            lax.fori_loop(0, reps, body, 0)

        s_ids(0)
        s_ids(1)
        descs["ids", 0].wait()
        add_offs(0)
        s_gat(0)
        if n_chunks > 2:
            s_ids(2)
        for i in range(n_chunks):
            if i + 1 < n_chunks:
                descs["ids", i + 1].wait()
                add_offs(i + 1)
                if i >= 1:
                    descs["out", i - 1].wait()
                s_gat(i + 1)
                if i + 3 < n_chunks:
                    s_ids(i + 3)
            descs["gat", i].wait()
            add_bias(i)
            s_out(i)
        if n_chunks >= 2:
            descs["out", n_chunks - 2].wait()
        descs["out", n_chunks - 1].wait()

    return run, chunk


def kernel(cat_ids, cond_table, cat_table):
    B, n_cond = cat_ids.shape
    n_rows, dim = cat_table.shape
    n_cat = n_rows // n_cond
    run, chunk = _build(B, n_cond, dim, n_cat)
    ids_flat = cat_ids.reshape(-1).astype(jnp.int32)
    offs = jnp.tile(jnp.arange(n_cond, dtype=jnp.int32) * n_cat,
                    chunk // n_cond)
    out = run(ids_flat, cond_table, cat_table, offs)
    return out.reshape(B, n_cond, dim)
